# Initial kernel scaffold; baseline (speedup 1.0000x reference)
import jax
import jax.numpy as jnp
from jax import lax
from jax.experimental import pallas as pl
from jax.experimental.pallas import tpu as pltpu
from jax.experimental.pallas import tpu_sc as plsc

_NUM_CORES = 2
_NUM_SUBCORES = 16
_NW = _NUM_CORES * _NUM_SUBCORES
_LANES = 16
_IDXW = 128


def _make_kernel(n_tokens, vocab, d, chunk):
    per_w = n_tokens // _NW
    n_chunks = per_w // chunk
    n_groups = chunk // _LANES
    n_streams = chunk // _IDXW
    groups_per_row = _IDXW // _LANES
    mesh = plsc.VectorSubcoreMesh(core_axis_name="c", subcore_axis_name="s")

    def body(ids_hbm, img_hbm, table_hbm, out_hbm,
             ids_v, nidx_v, vis_v, buf_v, img16_v, sem):
        wid = lax.axis_index("s") * _NUM_CORES + lax.axis_index("c")
        w_base = wid * per_w

        def chunk_body(k, carry):
            base = w_base + k * chunk
            pltpu.sync_copy(ids_hbm.at[pl.ds(base, chunk)], ids_v)

            def pass1(g, c):
                id16 = ids_v[pl.ds(g * _LANES, _LANES)]
                m = id16 >= vocab
                nid = jnp.where(m, vocab - 1, id16)
                r = g // groups_per_row
                co = (g % groups_per_row) * _LANES
                nidx_v[r, pl.ds(co, _LANES)] = nid
                return c

            lax.fori_loop(0, n_groups, pass1, 0)

            handles = [
                pltpu.async_copy(
                    table_hbm.at[nidx_v.at[j]],
                    buf_v.at[pl.ds(j * _IDXW, _IDXW)],
                    sem,
                )
                for j in range(n_streams)
            ]
            for h in handles:
                h.wait()

            def pass2(g, c):
                id16 = ids_v[pl.ds(g * _LANES, _LANES)]
                m = id16 >= vocab

                @pl.when(jnp.any(m))
                def _():
                    vis_v[...] = jnp.where(m, id16 - vocab, 0)
                    pltpu.async_copy(img_hbm.at[vis_v], img16_v, sem).wait()
                    lane = lax.broadcasted_iota(jnp.int32, (_LANES,), 0)
                    pos16 = lane + g * _LANES

                    def col(cc, c2):
                        colv = jnp.full((_LANES,), 0, jnp.int32) + cc
                        vals = plsc.load_gather(img16_v, [lane, colv])
                        plsc.store_scatter(buf_v, [pos16, colv], vals, mask=m)
                        return c2

                    lax.fori_loop(0, d, col, 0)

                return c

            lax.fori_loop(0, n_groups, pass2, 0)

            pltpu.sync_copy(buf_v, out_hbm.at[pl.ds(base, chunk)])
            return carry

        lax.fori_loop(0, n_chunks, chunk_body, 0)

    return pl.kernel(
        body,
        out_type=jax.ShapeDtypeStruct((n_tokens, d), jnp.float32),
        mesh=mesh,
        scratch_types=[
            pltpu.VMEM((chunk,), jnp.int32),
            pltpu.VMEM((n_streams, _IDXW), jnp.int32),
            pltpu.VMEM((_LANES,), jnp.int32),
            pltpu.VMEM((chunk, d), jnp.float32),
            pltpu.VMEM((_LANES, d), jnp.float32),
            pltpu.SemaphoreType.DMA,
        ],
    )


@jax.jit
def kernel(input_ids, image_embeds, table):
    b, s = input_ids.shape
    vocab, d = table.shape
    ids = input_ids.reshape(-1).astype(jnp.int32)
    n = ids.shape[0]
    k = _make_kernel(n, vocab, d, 512)
    out = k(ids, image_embeds, table)
    return out.reshape(b, s, d)


# speedup vs baseline: 2.1526x; 2.1526x over previous
import jax
import jax.numpy as jnp
from jax import lax
from jax.experimental import pallas as pl
from jax.experimental.pallas import tpu as pltpu
from jax.experimental.pallas import tpu_sc as plsc

_NUM_CORES = 2
_NUM_SUBCORES = 16
_NW = _NUM_CORES * _NUM_SUBCORES
_IDXW = 128


def _make_kernel(n_tokens, d, chunk):
    per_w = n_tokens // _NW
    n_chunks = per_w // chunk
    n_streams = chunk // _IDXW
    mesh = plsc.VectorSubcoreMesh(core_axis_name="c", subcore_axis_name="s")

    def body(ids_hbm, comb_hbm, out_hbm, idx_v, buf_v, sem):
        wid = lax.axis_index("s") * _NUM_CORES + lax.axis_index("c")
        w_base = wid * per_w

        def chunk_body(k, carry):
            base = w_base + k * chunk
            pltpu.sync_copy(
                ids_hbm.at[pl.ds(base // _IDXW, n_streams)], idx_v)
            handles = [
                pltpu.async_copy(
                    comb_hbm.at[idx_v.at[j]],
                    buf_v.at[pl.ds(j * _IDXW, _IDXW)],
                    sem,
                )
                for j in range(n_streams)
            ]
            for h in handles:
                h.wait()
            pltpu.sync_copy(buf_v, out_hbm.at[pl.ds(base, chunk)])
            return carry

        lax.fori_loop(0, n_chunks, chunk_body, 0)

    return pl.kernel(
        body,
        out_type=jax.ShapeDtypeStruct((n_tokens, d), jnp.float32),
        mesh=mesh,
        compiler_params=pltpu.CompilerParams(use_tc_tiling_on_sc=False),
        scratch_types=[
            pltpu.VMEM((n_streams, _IDXW), jnp.int32),
            pltpu.VMEM((chunk, d), jnp.float32),
            pltpu.SemaphoreType.DMA,
        ],
    )


@jax.jit
def kernel(input_ids, image_embeds, table):
    b, s = input_ids.shape
    d = table.shape[1]
    ids = input_ids.reshape(-1, _IDXW).astype(jnp.int32)
    combined = jnp.concatenate([table, image_embeds.astype(table.dtype)], axis=0)
    k = _make_kernel(ids.size, d, 512)
    out = k(ids, combined)
    return out.reshape(b, s, d)


# speedup vs baseline: 3.3004x; 1.5332x over previous
import jax
import jax.numpy as jnp
from jax import lax
from jax.experimental import pallas as pl
from jax.experimental.pallas import tpu as pltpu
from jax.experimental.pallas import tpu_sc as plsc

_NUM_CORES = 2
_NUM_SUBCORES = 16
_NW = _NUM_CORES * _NUM_SUBCORES
_IDXW = 128
_PADW = 128


def _make_kernel(n_tokens, d, chunk):
    per_w = n_tokens // _NW
    n_chunks = per_w // chunk
    n_streams = chunk // _IDXW
    mesh = plsc.VectorSubcoreMesh(core_axis_name="c", subcore_axis_name="s")

    idrows = 2 * n_streams

    def body(ids_hbm, comb_hbm, out_hbm, idx_v, buf_v, sem):
        wid = lax.axis_index("s") * _NUM_CORES + lax.axis_index("c")
        w_base = wid * per_w

        def chunk_body(k, carry):
            base = w_base + k * (2 * chunk)
            idrow0 = pl.multiple_of(base // _IDXW, 8)
            pltpu.sync_copy(
                ids_hbm.at[pl.ds(idrow0, idrows)], idx_v)
            for half in range(2):
                handles = [
                    pltpu.async_copy(
                        comb_hbm.at[idx_v.at[half * n_streams + j]],
                        buf_v.at[pl.ds(j * _IDXW, _IDXW)],
                        sem,
                    )
                    for j in range(n_streams)
                ]
                for h in handles:
                    h.wait()
                orow0 = pl.multiple_of(base + half * chunk, 8)
                pltpu.sync_copy(buf_v, out_hbm.at[pl.ds(orow0, chunk)])
            return carry

        lax.fori_loop(0, n_chunks // 2, chunk_body, 0)

    return pl.kernel(
        body,
        out_type=jax.ShapeDtypeStruct((n_tokens, _PADW), jnp.float32),
        mesh=mesh,
        scratch_types=[
            pltpu.VMEM((idrows, _IDXW), jnp.int32),
            pltpu.VMEM((chunk, _PADW), jnp.float32),
            pltpu.SemaphoreType.DMA,
        ],
    )


@jax.jit
def kernel(input_ids, image_embeds, table):
    b, s = input_ids.shape
    d = table.shape[1]
    ids = input_ids.reshape(-1, _IDXW).astype(jnp.int32)
    combined = jnp.concatenate([table, image_embeds.astype(table.dtype)], axis=0)
    combined = jnp.pad(combined, ((0, 0), (0, _PADW - d)))
    k = _make_kernel(ids.size, d, 512)
    out = k(ids, combined)
    return out[:, :d].reshape(b, s, d)


# speedup vs baseline: 3.3998x; 1.0301x over previous
import jax
import jax.numpy as jnp
from jax import lax
from jax.experimental import pallas as pl
from jax.experimental.pallas import tpu as pltpu
from jax.experimental.pallas import tpu_sc as plsc

_NUM_CORES = 2
_NUM_SUBCORES = 16
_NW = _NUM_CORES * _NUM_SUBCORES
_IDXW = 128
_PADW = 128
_CHUNK = 256
_BLOCK = 1024


def _make_kernel(n_tokens, d):
    per_w = n_tokens // _NW
    n_blocks = per_w // _BLOCK
    mesh = plsc.VectorSubcoreMesh(core_axis_name="c", subcore_axis_name="s")

    def body(ids_hbm, comb_hbm, out_hbm, idx_v, buf_a, buf_b, sem_a, sem_b):
        wid = lax.axis_index("s") * _NUM_CORES + lax.axis_index("c")
        w_base = wid * per_w

        def fire(buf, sem, rows):
            return [
                pltpu.async_copy(
                    comb_hbm.at[idx_v.at[rows[j]]],
                    buf.at[pl.ds(j * _IDXW, _IDXW)],
                    sem,
                )
                for j in range(len(rows))
            ]

        def block_body(k, carry):
            base = w_base + k * _BLOCK
            idrow0 = pl.multiple_of(base // _IDXW, 8)
            pltpu.sync_copy(ids_hbm.at[pl.ds(idrow0, 8)], idx_v)

            bufs = (buf_a, buf_b)
            sems = (sem_a, sem_b)
            pending = [None, None]
            handles = [None, None]
            n_chunks = _BLOCK // _CHUNK
            rows_per_chunk = _CHUNK // _IDXW
            for c in range(n_chunks):
                p = c % 2
                rows = [c * rows_per_chunk + j for j in range(rows_per_chunk)]
                if handles[p] is not None:
                    for h in handles[p]:
                        h.wait()
                    orow = pl.multiple_of(
                        base + pending[p] * _CHUNK, 8)
                    pltpu.sync_copy(bufs[p], out_hbm.at[pl.ds(orow, _CHUNK)])
                handles[p] = fire(bufs[p], sems[p], rows)
                pending[p] = c
            for p in (0, 1):
                if handles[p] is not None:
                    for h in handles[p]:
                        h.wait()
                    orow = pl.multiple_of(base + pending[p] * _CHUNK, 8)
                    pltpu.sync_copy(bufs[p], out_hbm.at[pl.ds(orow, _CHUNK)])
            return carry

        lax.fori_loop(0, n_blocks, block_body, 0)

    return pl.kernel(
        body,
        out_type=jax.ShapeDtypeStruct((n_tokens, _PADW), jnp.float32),
        mesh=mesh,
        scratch_types=[
            pltpu.VMEM((8, _IDXW), jnp.int32),
            pltpu.VMEM((_CHUNK, _PADW), jnp.float32),
            pltpu.VMEM((_CHUNK, _PADW), jnp.float32),
            pltpu.SemaphoreType.DMA,
            pltpu.SemaphoreType.DMA,
        ],
    )


@jax.jit
def kernel(input_ids, image_embeds, table):
    b, s = input_ids.shape
    d = table.shape[1]
    ids = input_ids.reshape(-1, _IDXW).astype(jnp.int32)
    combined = jnp.concatenate([table, image_embeds.astype(table.dtype)], axis=0)
    combined = jnp.pad(combined, ((0, 0), (0, _PADW - d)))
    k = _make_kernel(ids.size, d)
    out = k(ids, combined)
    return out[:, :d].reshape(b, s, d)
